# Initial kernel scaffold; baseline (speedup 1.0000x reference)
import jax
import jax.numpy as jnp
from jax.experimental import pallas as pl


def kernel(x, stem_w, stem_b, s0b0_w1, s0b0_b1, s0b0_w2, s0b0_b2, s0b0_w3, s0b0_b3, s0b0_wd, s0b0_bd, s0b1_w1, s0b1_b1, s0b1_w2, s0b1_b2, s0b1_w3, s0b1_b3, s0b2_w1, s0b2_b1, s0b2_w2, s0b2_b2, s0b2_w3, s0b2_b3, s1b0_w1, s1b0_b1, s1b0_w2, s1b0_b2, s1b0_w3, s1b0_b3, s1b0_wd, s1b0_bd, s1b1_w1, s1b1_b1, s1b1_w2, s1b1_b2, s1b1_w3, s1b1_b3, s1b2_w1, s1b2_b1, s1b2_w2, s1b2_b2, s1b2_w3, s1b2_b3, s1b3_w1, s1b3_b1, s1b3_w2, s1b3_b2, s1b3_w3, s1b3_b3, s2b0_w1, s2b0_b1, s2b0_w2, s2b0_b2, s2b0_w3, s2b0_b3, s2b0_wd, s2b0_bd, s2b1_w1, s2b1_b1, s2b1_w2, s2b1_b2, s2b1_w3, s2b1_b3, s2b2_w1, s2b2_b1, s2b2_w2, s2b2_b2, s2b2_w3, s2b2_b3, s2b3_w1, s2b3_b1, s2b3_w2, s2b3_b2, s2b3_w3, s2b3_b3, s2b4_w1, s2b4_b1, s2b4_w2, s2b4_b2, s2b4_w3, s2b4_b3, s2b5_w1, s2b5_b1, s2b5_w2, s2b5_b2, s2b5_w3, s2b5_b3, s3b0_w1, s3b0_b1, s3b0_w2, s3b0_b2, s3b0_w3, s3b0_b3, s3b0_wd, s3b0_bd, s3b1_w1, s3b1_b1, s3b1_w2, s3b1_b2, s3b1_w3, s3b1_b3, s3b2_w1, s3b2_b1, s3b2_w2, s3b2_b2, s3b2_w3, s3b2_b3, fc1_w, fc1_b, fc2_w, fc2_b, hbn1_s, hbn1_h, hbn2_s, hbn2_h):
    raise NotImplementedError("write your pallas kernel here")


# speedup vs baseline: 1.0301x; 1.0301x over previous
import jax
import jax.numpy as jnp
from jax.experimental import pallas as pl
from jax.experimental.pallas import tpu as pltpu

_GS = 4
_G = 2


def _halo(w):
    return ((w + 8) // 8) * 8


def _conv3x3(h1, w2_ref, b2, h, w, scr):
    m, p = h1.shape
    lw = w.bit_length() - 1
    base = _halo(w)
    scr[pl.ds(base, m), :] = h1
    row = jax.lax.broadcasted_iota(jnp.int32, (m, p), 0)
    ii = (row >> lw) & (h - 1)
    jj = row & (w - 1)
    acc = jnp.broadcast_to(b2, (m, p)).astype(jnp.float32)
    for di in (-1, 0, 1):
        for dj in (-1, 0, 1):
            t = (di + 1) * 3 + (dj + 1)
            if di == 0 and dj == 0:
                tap = h1
            else:
                tap = scr[pl.ds(base + di * w + dj, m), :]
                keep = (((ii + di) >= 0) & ((ii + di) < h) &
                        ((jj + dj) >= 0) & ((jj + dj) < w))
                tap = jnp.where(keep, tap, 0.0)
            acc = acc + jnp.dot(tap.astype(jnp.bfloat16),
                                w2_ref[pl.ds(t * p, p), :],
                                preferred_element_type=jnp.float32)
    return acc


def _downsel(m_in, h, w):
    ho, wo = h // 2, w // 2
    n_img = m_in // (h * w)
    m_out = n_img * ho * wo
    lw, lwo, lho = w.bit_length() - 1, wo.bit_length() - 1, ho.bit_length() - 1
    ro = jax.lax.broadcasted_iota(jnp.int32, (m_out, m_in), 0)
    ci = jax.lax.broadcasted_iota(jnp.int32, (m_out, m_in), 1)
    img = ro >> (lho + lwo)
    oi = (ro >> lwo) & (ho - 1)
    oj = ro & (wo - 1)
    src = img * (h * w) + (oi << (lw + 1)) + (oj << 1)
    return (ci == src).astype(jnp.bfloat16)


def _bottleneck(xv, wr, h, stride, has_ds, scr):
    f32, bf16 = jnp.float32, jnp.bfloat16
    w1, b1, w2, b2, w3, b3 = wr[:6]
    h1 = jnp.maximum(jnp.dot(xv, w1[...], preferred_element_type=f32) + b1[...], 0.0)
    h2 = jnp.maximum(_conv3x3(h1, w2, b2[...], h, h, scr), 0.0)
    if stride == 2:
        sel = _downsel(xv.shape[0], h, h)
        h2 = jnp.dot(sel, h2.astype(bf16), preferred_element_type=f32)
        xid = jnp.dot(sel, xv, preferred_element_type=f32).astype(bf16)
    else:
        xid = xv
    z = jnp.dot(h2.astype(bf16), w3[...], preferred_element_type=f32) + b3[...]
    if has_ds:
        z = z + jnp.dot(xid, wr[6][...], preferred_element_type=f32) + wr[7][...]
    else:
        z = z + xid.astype(f32)
    return jnp.maximum(z, 0.0).astype(bf16)


_STAGE_DESCS = [
    (16, [(1, True), (1, False), (1, False)]),
    (16, [(2, True), (1, False), (1, False), (1, False)]),
    (8,  [(2, True), (1, False), (1, False), (1, False), (1, False), (1, False)]),
    (4,  [(2, True), (1, False), (1, False)]),
]
_N_WREFS = sum(8 if ds else 6 for _, blks in _STAGE_DESCS for _, ds in blks)


def _stages_body(*refs):
    x_ref = refs[0]
    out_ref = refs[1 + _N_WREFS]
    scrs = refs[2 + _N_WREFS: 6 + _N_WREFS]
    for s in scrs:
        s[...] = jnp.zeros(s.shape, s.dtype)
    cur = x_ref[...]
    k = 1
    for si, (h0, blks) in enumerate(_STAGE_DESCS):
        h = h0
        for stride, has_ds in blks:
            cnt = 8 if has_ds else 6
            cur = _bottleneck(cur, refs[k:k + cnt], h, stride, has_ds, scrs[si])
            k += cnt
            h //= stride
    out_ref[...] = cur


def _stem_body(p_ref, w_ref, b_ref, o_ref):
    z = jnp.dot(p_ref[...], w_ref[...], preferred_element_type=jnp.float32) + b_ref[...]
    o_ref[...] = jnp.maximum(z, 0.0).astype(o_ref.dtype)


def _head_body(a_ref, w1_ref, b1_ref, s1_ref, t1_ref,
               w2_ref, b2_ref, s2_ref, t2_ref, o_ref):
    f32 = jnp.float32
    z1 = jnp.dot(a_ref[...], w1_ref[...], preferred_element_type=f32) + b1_ref[...]
    y1 = jnp.maximum(z1 * s1_ref[...] + t1_ref[...], 0.0)
    z2 = jnp.dot(y1.astype(jnp.bfloat16), w2_ref[...],
                 preferred_element_type=f32) + b2_ref[...]
    o_ref[...] = jnp.maximum(z2 * s2_ref[...] + t2_ref[...], 0.0)


def kernel(x, stem_w, stem_b, s0b0_w1, s0b0_b1, s0b0_w2, s0b0_b2, s0b0_w3, s0b0_b3, s0b0_wd, s0b0_bd, s0b1_w1, s0b1_b1, s0b1_w2, s0b1_b2, s0b1_w3, s0b1_b3, s0b2_w1, s0b2_b1, s0b2_w2, s0b2_b2, s0b2_w3, s0b2_b3, s1b0_w1, s1b0_b1, s1b0_w2, s1b0_b2, s1b0_w3, s1b0_b3, s1b0_wd, s1b0_bd, s1b1_w1, s1b1_b1, s1b1_w2, s1b1_b2, s1b1_w3, s1b1_b3, s1b2_w1, s1b2_b1, s1b2_w2, s1b2_b2, s1b2_w3, s1b2_b3, s1b3_w1, s1b3_b1, s1b3_w2, s1b3_b2, s1b3_w3, s1b3_b3, s2b0_w1, s2b0_b1, s2b0_w2, s2b0_b2, s2b0_w3, s2b0_b3, s2b0_wd, s2b0_bd, s2b1_w1, s2b1_b1, s2b1_w2, s2b1_b2, s2b1_w3, s2b1_b3, s2b2_w1, s2b2_b1, s2b2_w2, s2b2_b2, s2b2_w3, s2b2_b3, s2b3_w1, s2b3_b1, s2b3_w2, s2b3_b2, s2b3_w3, s2b3_b3, s2b4_w1, s2b4_b1, s2b4_w2, s2b4_b2, s2b4_w3, s2b4_b3, s2b5_w1, s2b5_b1, s2b5_w2, s2b5_b2, s2b5_w3, s2b5_b3, s3b0_w1, s3b0_b1, s3b0_w2, s3b0_b2, s3b0_w3, s3b0_b3, s3b0_wd, s3b0_bd, s3b1_w1, s3b1_b1, s3b1_w2, s3b1_b2, s3b1_w3, s3b1_b3, s3b2_w1, s3b2_b1, s3b2_w2, s3b2_b2, s3b2_w3, s3b2_b3, fc1_w, fc1_b, fc2_w, fc2_b, hbn1_s, hbn1_h, hbn2_s, hbn2_h):
    n = x.shape[0]
    f32, bf16 = jnp.float32, jnp.bfloat16

    xh = jnp.transpose(x, (0, 2, 3, 1)).astype(bf16)
    xp = jnp.pad(xh, ((0, 0), (3, 3), (3, 3), (0, 0)))
    cols = [xp[:, di:di + 64:2, dj:dj + 64:2, :]
            for di in range(7) for dj in range(7)]
    patches = jnp.concatenate(cols, axis=-1).reshape(n * 1024, 147)

    gs = _GS if n % _GS == 0 else 1
    stem = pl.pallas_call(
        _stem_body,
        out_shape=jax.ShapeDtypeStruct((n * 1024, 128), bf16),
        grid=(n // gs,),
        in_specs=[pl.BlockSpec((gs * 1024, 147), lambda i: (i, 0)),
                  pl.BlockSpec(stem_w.shape, lambda i: (0, 0)),
                  pl.BlockSpec(stem_b.shape, lambda i: (0, 0))],
        out_specs=pl.BlockSpec((gs * 1024, 128), lambda i: (i, 0)),
        compiler_params=pltpu.CompilerParams(dimension_semantics=("parallel",)),
    )(patches, stem_w, stem_b)

    sm = stem.reshape(n, 32, 32, 128)
    smp = jnp.pad(sm, ((0, 0), (1, 1), (1, 1), (0, 0)),
                  constant_values=-jnp.inf)
    feat = None
    for di in range(3):
        for dj in range(3):
            s = smp[:, di:di + 32:2, dj:dj + 32:2, :]
            feat = s if feat is None else jnp.maximum(feat, s)
    feat = feat.reshape(n * 256, 128)

    wargs = [s0b0_w1, s0b0_b1, s0b0_w2, s0b0_b2, s0b0_w3, s0b0_b3, s0b0_wd, s0b0_bd,
             s0b1_w1, s0b1_b1, s0b1_w2, s0b1_b2, s0b1_w3, s0b1_b3,
             s0b2_w1, s0b2_b1, s0b2_w2, s0b2_b2, s0b2_w3, s0b2_b3,
             s1b0_w1, s1b0_b1, s1b0_w2, s1b0_b2, s1b0_w3, s1b0_b3, s1b0_wd, s1b0_bd,
             s1b1_w1, s1b1_b1, s1b1_w2, s1b1_b2, s1b1_w3, s1b1_b3,
             s1b2_w1, s1b2_b1, s1b2_w2, s1b2_b2, s1b2_w3, s1b2_b3,
             s1b3_w1, s1b3_b1, s1b3_w2, s1b3_b2, s1b3_w3, s1b3_b3,
             s2b0_w1, s2b0_b1, s2b0_w2, s2b0_b2, s2b0_w3, s2b0_b3, s2b0_wd, s2b0_bd,
             s2b1_w1, s2b1_b1, s2b1_w2, s2b1_b2, s2b1_w3, s2b1_b3,
             s2b2_w1, s2b2_b1, s2b2_w2, s2b2_b2, s2b2_w3, s2b2_b3,
             s2b3_w1, s2b3_b1, s2b3_w2, s2b3_b2, s2b3_w3, s2b3_b3,
             s2b4_w1, s2b4_b1, s2b4_w2, s2b4_b2, s2b4_w3, s2b4_b3,
             s2b5_w1, s2b5_b1, s2b5_w2, s2b5_b2, s2b5_w3, s2b5_b3,
             s3b0_w1, s3b0_b1, s3b0_w2, s3b0_b2, s3b0_w3, s3b0_b3, s3b0_wd, s3b0_bd,
             s3b1_w1, s3b1_b1, s3b1_w2, s3b1_b2, s3b1_w3, s3b1_b3,
             s3b2_w1, s3b2_b1, s3b2_w2, s3b2_b2, s3b2_w3, s3b2_b3]

    g = _G if n % _G == 0 else 1
    scratch = [pltpu.VMEM((g * 256 + 2 * _halo(16), 8), f32),
               pltpu.VMEM((g * 256 + 2 * _halo(16), 16), f32),
               pltpu.VMEM((g * 64 + 2 * _halo(8), 32), f32),
               pltpu.VMEM((g * 16 + 2 * _halo(4), 64), f32)]
    res = pl.pallas_call(
        _stages_body,
        out_shape=jax.ShapeDtypeStruct((n * 4, 256), bf16),
        grid=(n // g,),
        in_specs=[pl.BlockSpec((g * 256, 128), lambda i: (i, 0))] +
                 [pl.BlockSpec(a.shape, lambda i: (0, 0)) for a in wargs],
        out_specs=pl.BlockSpec((g * 4, 256), lambda i: (i, 0)),
        scratch_shapes=scratch,
        compiler_params=pltpu.CompilerParams(dimension_semantics=("parallel",)),
    )(feat, *wargs)

    a = res.reshape(n, 2, 2, 256).transpose(0, 3, 1, 2).reshape(n * 4, 256)
    s1 = jnp.tile(hbn1_s, n).reshape(n * 4, 1).astype(f32)
    t1 = jnp.tile(hbn1_h, n).reshape(n * 4, 1).astype(f32)
    s2 = jnp.tile(hbn2_s, n).reshape(n * 4, 1).astype(f32)
    t2 = jnp.tile(hbn2_h, n).reshape(n * 4, 1).astype(f32)

    rows = n * 4
    gh = 2 if rows % 16 == 0 else 1
    rb = rows // gh
    out = pl.pallas_call(
        _head_body,
        out_shape=jax.ShapeDtypeStruct((rows, 128), f32),
        grid=(gh,),
        in_specs=[pl.BlockSpec((rb, 256), lambda i: (i, 0)),
                  pl.BlockSpec(fc1_w.shape, lambda i: (0, 0)),
                  pl.BlockSpec(fc1_b.shape, lambda i: (0, 0)),
                  pl.BlockSpec((rb, 1), lambda i: (i, 0)),
                  pl.BlockSpec((rb, 1), lambda i: (i, 0)),
                  pl.BlockSpec(fc2_w.shape, lambda i: (0, 0)),
                  pl.BlockSpec(fc2_b.shape, lambda i: (0, 0)),
                  pl.BlockSpec((rb, 1), lambda i: (i, 0)),
                  pl.BlockSpec((rb, 1), lambda i: (i, 0))],
        out_specs=pl.BlockSpec((rb, 128), lambda i: (i, 0)),
        compiler_params=pltpu.CompilerParams(dimension_semantics=("parallel",)),
    )(a, fc1_w, fc1_b, s1, t1, fc2_w, fc2_b, s2, t2)

    return out[:, :32].reshape(n, 4, 32)


# speedup vs baseline: 1.0516x; 1.0208x over previous
import jax
import jax.numpy as jnp
from jax.experimental import pallas as pl
from jax.experimental.pallas import tpu as pltpu

_GS = 8
_G = 4


def _halo(w):
    return ((w + 8) // 8) * 8


def _conv3x3(h1, w2_ref, b2, h, w, scr):
    m, p = h1.shape
    lw = w.bit_length() - 1
    base = _halo(w)
    scr[pl.ds(base, m), :] = h1
    row = jax.lax.broadcasted_iota(jnp.int32, (m, p), 0)
    ii = (row >> lw) & (h - 1)
    jj = row & (w - 1)
    acc = jnp.broadcast_to(b2, (m, p)).astype(jnp.float32)
    for di in (-1, 0, 1):
        for dj in (-1, 0, 1):
            t = (di + 1) * 3 + (dj + 1)
            if di == 0 and dj == 0:
                tap = h1
            else:
                tap = scr[pl.ds(base + di * w + dj, m), :]
                keep = (((ii + di) >= 0) & ((ii + di) < h) &
                        ((jj + dj) >= 0) & ((jj + dj) < w))
                tap = jnp.where(keep, tap, 0.0)
            acc = acc + jnp.dot(tap.astype(jnp.bfloat16),
                                w2_ref[pl.ds(t * p, p), :],
                                preferred_element_type=jnp.float32)
    return acc


def _downsel(m_in, h, w):
    ho, wo = h // 2, w // 2
    n_img = m_in // (h * w)
    m_out = n_img * ho * wo
    lw, lwo, lho = w.bit_length() - 1, wo.bit_length() - 1, ho.bit_length() - 1
    ro = jax.lax.broadcasted_iota(jnp.int32, (m_out, m_in), 0)
    ci = jax.lax.broadcasted_iota(jnp.int32, (m_out, m_in), 1)
    img = ro >> (lho + lwo)
    oi = (ro >> lwo) & (ho - 1)
    oj = ro & (wo - 1)
    src = img * (h * w) + (oi << (lw + 1)) + (oj << 1)
    return (ci == src).astype(jnp.bfloat16)


def _bottleneck(xv, wr, h, stride, has_ds, scr):
    f32, bf16 = jnp.float32, jnp.bfloat16
    w1, b1, w2, b2, w3, b3 = wr[:6]
    h1 = jnp.maximum(jnp.dot(xv, w1[...], preferred_element_type=f32) + b1[...], 0.0)
    h2 = jnp.maximum(_conv3x3(h1, w2, b2[...], h, h, scr), 0.0)
    if stride == 2:
        sel = _downsel(xv.shape[0], h, h)
        h2 = jnp.dot(sel, h2.astype(bf16), preferred_element_type=f32)
        xid = jnp.dot(sel, xv, preferred_element_type=f32).astype(bf16)
    else:
        xid = xv
    z = jnp.dot(h2.astype(bf16), w3[...], preferred_element_type=f32) + b3[...]
    if has_ds:
        z = z + jnp.dot(xid, wr[6][...], preferred_element_type=f32) + wr[7][...]
    else:
        z = z + xid.astype(f32)
    return jnp.maximum(z, 0.0).astype(bf16)


_STAGE_DESCS = [
    (16, [(1, True), (1, False), (1, False)]),
    (16, [(2, True), (1, False), (1, False), (1, False)]),
    (8,  [(2, True), (1, False), (1, False), (1, False), (1, False), (1, False)]),
    (4,  [(2, True), (1, False), (1, False)]),
]
_N_WREFS = sum(8 if ds else 6 for _, blks in _STAGE_DESCS for _, ds in blks)


def _stages_body(*refs):
    x_ref = refs[0]
    out_ref = refs[1 + _N_WREFS]
    scrs = refs[2 + _N_WREFS: 6 + _N_WREFS]
    for s in scrs:
        s[...] = jnp.zeros(s.shape, s.dtype)
    cur = x_ref[...]
    k = 1
    for si, (h0, blks) in enumerate(_STAGE_DESCS):
        h = h0
        for stride, has_ds in blks:
            cnt = 8 if has_ds else 6
            cur = _bottleneck(cur, refs[k:k + cnt], h, stride, has_ds, scrs[si])
            k += cnt
            h //= stride
    out_ref[...] = cur


def _stem_body(p_ref, w_ref, b_ref, o_ref):
    z = jnp.dot(p_ref[...], w_ref[...], preferred_element_type=jnp.float32) + b_ref[...]
    o_ref[...] = jnp.maximum(z, 0.0).astype(o_ref.dtype)


def _head_body(a_ref, w1_ref, b1_ref, s1_ref, t1_ref,
               w2_ref, b2_ref, s2_ref, t2_ref, o_ref):
    f32 = jnp.float32
    z1 = jnp.dot(a_ref[...], w1_ref[...], preferred_element_type=f32) + b1_ref[...]
    y1 = jnp.maximum(z1 * s1_ref[...] + t1_ref[...], 0.0)
    z2 = jnp.dot(y1.astype(jnp.bfloat16), w2_ref[...],
                 preferred_element_type=f32) + b2_ref[...]
    o_ref[...] = jnp.maximum(z2 * s2_ref[...] + t2_ref[...], 0.0)


def kernel(x, stem_w, stem_b, s0b0_w1, s0b0_b1, s0b0_w2, s0b0_b2, s0b0_w3, s0b0_b3, s0b0_wd, s0b0_bd, s0b1_w1, s0b1_b1, s0b1_w2, s0b1_b2, s0b1_w3, s0b1_b3, s0b2_w1, s0b2_b1, s0b2_w2, s0b2_b2, s0b2_w3, s0b2_b3, s1b0_w1, s1b0_b1, s1b0_w2, s1b0_b2, s1b0_w3, s1b0_b3, s1b0_wd, s1b0_bd, s1b1_w1, s1b1_b1, s1b1_w2, s1b1_b2, s1b1_w3, s1b1_b3, s1b2_w1, s1b2_b1, s1b2_w2, s1b2_b2, s1b2_w3, s1b2_b3, s1b3_w1, s1b3_b1, s1b3_w2, s1b3_b2, s1b3_w3, s1b3_b3, s2b0_w1, s2b0_b1, s2b0_w2, s2b0_b2, s2b0_w3, s2b0_b3, s2b0_wd, s2b0_bd, s2b1_w1, s2b1_b1, s2b1_w2, s2b1_b2, s2b1_w3, s2b1_b3, s2b2_w1, s2b2_b1, s2b2_w2, s2b2_b2, s2b2_w3, s2b2_b3, s2b3_w1, s2b3_b1, s2b3_w2, s2b3_b2, s2b3_w3, s2b3_b3, s2b4_w1, s2b4_b1, s2b4_w2, s2b4_b2, s2b4_w3, s2b4_b3, s2b5_w1, s2b5_b1, s2b5_w2, s2b5_b2, s2b5_w3, s2b5_b3, s3b0_w1, s3b0_b1, s3b0_w2, s3b0_b2, s3b0_w3, s3b0_b3, s3b0_wd, s3b0_bd, s3b1_w1, s3b1_b1, s3b1_w2, s3b1_b2, s3b1_w3, s3b1_b3, s3b2_w1, s3b2_b1, s3b2_w2, s3b2_b2, s3b2_w3, s3b2_b3, fc1_w, fc1_b, fc2_w, fc2_b, hbn1_s, hbn1_h, hbn2_s, hbn2_h):
    n = x.shape[0]
    f32, bf16 = jnp.float32, jnp.bfloat16

    xh = jnp.transpose(x, (0, 2, 3, 1)).astype(bf16)
    xp = jnp.pad(xh, ((0, 0), (3, 3), (3, 3), (0, 0)))
    cols = [xp[:, di:di + 64:2, dj:dj + 64:2, :]
            for di in range(7) for dj in range(7)]
    patches = jnp.concatenate(cols, axis=-1).reshape(n * 1024, 147)

    gs = _GS if n % _GS == 0 else 1
    stem = pl.pallas_call(
        _stem_body,
        out_shape=jax.ShapeDtypeStruct((n * 1024, 128), bf16),
        grid=(n // gs,),
        in_specs=[pl.BlockSpec((gs * 1024, 147), lambda i: (i, 0)),
                  pl.BlockSpec(stem_w.shape, lambda i: (0, 0)),
                  pl.BlockSpec(stem_b.shape, lambda i: (0, 0))],
        out_specs=pl.BlockSpec((gs * 1024, 128), lambda i: (i, 0)),
        compiler_params=pltpu.CompilerParams(dimension_semantics=("parallel",)),
    )(patches, stem_w, stem_b)

    sm = stem.reshape(n, 32, 32, 128)
    smp = jnp.pad(sm, ((0, 0), (1, 1), (1, 1), (0, 0)),
                  constant_values=-jnp.inf)
    feat = None
    for di in range(3):
        for dj in range(3):
            s = smp[:, di:di + 32:2, dj:dj + 32:2, :]
            feat = s if feat is None else jnp.maximum(feat, s)
    feat = feat.reshape(n * 256, 128)

    wargs = [s0b0_w1, s0b0_b1, s0b0_w2, s0b0_b2, s0b0_w3, s0b0_b3, s0b0_wd, s0b0_bd,
             s0b1_w1, s0b1_b1, s0b1_w2, s0b1_b2, s0b1_w3, s0b1_b3,
             s0b2_w1, s0b2_b1, s0b2_w2, s0b2_b2, s0b2_w3, s0b2_b3,
             s1b0_w1, s1b0_b1, s1b0_w2, s1b0_b2, s1b0_w3, s1b0_b3, s1b0_wd, s1b0_bd,
             s1b1_w1, s1b1_b1, s1b1_w2, s1b1_b2, s1b1_w3, s1b1_b3,
             s1b2_w1, s1b2_b1, s1b2_w2, s1b2_b2, s1b2_w3, s1b2_b3,
             s1b3_w1, s1b3_b1, s1b3_w2, s1b3_b2, s1b3_w3, s1b3_b3,
             s2b0_w1, s2b0_b1, s2b0_w2, s2b0_b2, s2b0_w3, s2b0_b3, s2b0_wd, s2b0_bd,
             s2b1_w1, s2b1_b1, s2b1_w2, s2b1_b2, s2b1_w3, s2b1_b3,
             s2b2_w1, s2b2_b1, s2b2_w2, s2b2_b2, s2b2_w3, s2b2_b3,
             s2b3_w1, s2b3_b1, s2b3_w2, s2b3_b2, s2b3_w3, s2b3_b3,
             s2b4_w1, s2b4_b1, s2b4_w2, s2b4_b2, s2b4_w3, s2b4_b3,
             s2b5_w1, s2b5_b1, s2b5_w2, s2b5_b2, s2b5_w3, s2b5_b3,
             s3b0_w1, s3b0_b1, s3b0_w2, s3b0_b2, s3b0_w3, s3b0_b3, s3b0_wd, s3b0_bd,
             s3b1_w1, s3b1_b1, s3b1_w2, s3b1_b2, s3b1_w3, s3b1_b3,
             s3b2_w1, s3b2_b1, s3b2_w2, s3b2_b2, s3b2_w3, s3b2_b3]

    g = _G if n % _G == 0 else 1
    scratch = [pltpu.VMEM((g * 256 + 2 * _halo(16), 8), f32),
               pltpu.VMEM((g * 256 + 2 * _halo(16), 16), f32),
               pltpu.VMEM((g * 64 + 2 * _halo(8), 32), f32),
               pltpu.VMEM((g * 16 + 2 * _halo(4), 64), f32)]
    res = pl.pallas_call(
        _stages_body,
        out_shape=jax.ShapeDtypeStruct((n * 4, 256), bf16),
        grid=(n // g,),
        in_specs=[pl.BlockSpec((g * 256, 128), lambda i: (i, 0))] +
                 [pl.BlockSpec(a.shape, lambda i: (0, 0)) for a in wargs],
        out_specs=pl.BlockSpec((g * 4, 256), lambda i: (i, 0)),
        scratch_shapes=scratch,
        compiler_params=pltpu.CompilerParams(dimension_semantics=("parallel",)),
    )(feat, *wargs)

    a = res.reshape(n, 2, 2, 256).transpose(0, 3, 1, 2).reshape(n * 4, 256)
    s1 = jnp.tile(hbn1_s, n).reshape(n * 4, 1).astype(f32)
    t1 = jnp.tile(hbn1_h, n).reshape(n * 4, 1).astype(f32)
    s2 = jnp.tile(hbn2_s, n).reshape(n * 4, 1).astype(f32)
    t2 = jnp.tile(hbn2_h, n).reshape(n * 4, 1).astype(f32)

    rows = n * 4
    gh = 2 if rows % 16 == 0 else 1
    rb = rows // gh
    out = pl.pallas_call(
        _head_body,
        out_shape=jax.ShapeDtypeStruct((rows, 128), f32),
        grid=(gh,),
        in_specs=[pl.BlockSpec((rb, 256), lambda i: (i, 0)),
                  pl.BlockSpec(fc1_w.shape, lambda i: (0, 0)),
                  pl.BlockSpec(fc1_b.shape, lambda i: (0, 0)),
                  pl.BlockSpec((rb, 1), lambda i: (i, 0)),
                  pl.BlockSpec((rb, 1), lambda i: (i, 0)),
                  pl.BlockSpec(fc2_w.shape, lambda i: (0, 0)),
                  pl.BlockSpec(fc2_b.shape, lambda i: (0, 0)),
                  pl.BlockSpec((rb, 1), lambda i: (i, 0)),
                  pl.BlockSpec((rb, 1), lambda i: (i, 0))],
        out_specs=pl.BlockSpec((rb, 128), lambda i: (i, 0)),
        compiler_params=pltpu.CompilerParams(dimension_semantics=("parallel",)),
    )(a, fc1_w, fc1_b, s1, t1, fc2_w, fc2_b, s2, t2)

    return out[:, :32].reshape(n, 4, 32)


# speedup vs baseline: 1.0592x; 1.0073x over previous
import jax
import jax.numpy as jnp
from jax.experimental import pallas as pl
from jax.experimental.pallas import tpu as pltpu

_GS = 16
_G = 8


def _halo(w):
    return ((w + 8) // 8) * 8


def _conv3x3(h1, w2_ref, b2, h, w, scr):
    m, p = h1.shape
    lw = w.bit_length() - 1
    base = _halo(w)
    scr[pl.ds(base, m), :] = h1
    row = jax.lax.broadcasted_iota(jnp.int32, (m, p), 0)
    ii = (row >> lw) & (h - 1)
    jj = row & (w - 1)
    acc = jnp.broadcast_to(b2, (m, p)).astype(jnp.float32)
    for di in (-1, 0, 1):
        for dj in (-1, 0, 1):
            t = (di + 1) * 3 + (dj + 1)
            if di == 0 and dj == 0:
                tap = h1
            else:
                tap = scr[pl.ds(base + di * w + dj, m), :]
                keep = (((ii + di) >= 0) & ((ii + di) < h) &
                        ((jj + dj) >= 0) & ((jj + dj) < w))
                tap = jnp.where(keep, tap, 0.0)
            acc = acc + jnp.dot(tap.astype(jnp.bfloat16),
                                w2_ref[pl.ds(t * p, p), :],
                                preferred_element_type=jnp.float32)
    return acc


def _downsel(m_in, h, w):
    ho, wo = h // 2, w // 2
    n_img = m_in // (h * w)
    m_out = n_img * ho * wo
    lw, lwo, lho = w.bit_length() - 1, wo.bit_length() - 1, ho.bit_length() - 1
    ro = jax.lax.broadcasted_iota(jnp.int32, (m_out, m_in), 0)
    ci = jax.lax.broadcasted_iota(jnp.int32, (m_out, m_in), 1)
    img = ro >> (lho + lwo)
    oi = (ro >> lwo) & (ho - 1)
    oj = ro & (wo - 1)
    src = img * (h * w) + (oi << (lw + 1)) + (oj << 1)
    return (ci == src).astype(jnp.bfloat16)


def _bottleneck(xv, wr, h, stride, has_ds, scr):
    f32, bf16 = jnp.float32, jnp.bfloat16
    w1, b1, w2, b2, w3, b3 = wr[:6]
    h1 = jnp.maximum(jnp.dot(xv, w1[...], preferred_element_type=f32) + b1[...], 0.0)
    h2 = jnp.maximum(_conv3x3(h1, w2, b2[...], h, h, scr), 0.0)
    if stride == 2:
        sel = _downsel(xv.shape[0], h, h)
        h2 = jnp.dot(sel, h2.astype(bf16), preferred_element_type=f32)
        xid = jnp.dot(sel, xv, preferred_element_type=f32).astype(bf16)
    else:
        xid = xv
    z = jnp.dot(h2.astype(bf16), w3[...], preferred_element_type=f32) + b3[...]
    if has_ds:
        z = z + jnp.dot(xid, wr[6][...], preferred_element_type=f32) + wr[7][...]
    else:
        z = z + xid.astype(f32)
    return jnp.maximum(z, 0.0).astype(bf16)


_STAGE_DESCS = [
    (16, [(1, True), (1, False), (1, False)]),
    (16, [(2, True), (1, False), (1, False), (1, False)]),
    (8,  [(2, True), (1, False), (1, False), (1, False), (1, False), (1, False)]),
    (4,  [(2, True), (1, False), (1, False)]),
]
_N_WREFS = sum(8 if ds else 6 for _, blks in _STAGE_DESCS for _, ds in blks)


def _stages_body(*refs):
    x_ref = refs[0]
    out_ref = refs[1 + _N_WREFS]
    scrs = refs[2 + _N_WREFS: 6 + _N_WREFS]
    for s in scrs:
        s[...] = jnp.zeros(s.shape, s.dtype)
    cur = x_ref[...]
    k = 1
    for si, (h0, blks) in enumerate(_STAGE_DESCS):
        h = h0
        for stride, has_ds in blks:
            cnt = 8 if has_ds else 6
            cur = _bottleneck(cur, refs[k:k + cnt], h, stride, has_ds, scrs[si])
            k += cnt
            h //= stride
    out_ref[...] = cur


def _stem_body(p_ref, w_ref, b_ref, o_ref):
    z = jnp.dot(p_ref[...], w_ref[...], preferred_element_type=jnp.float32) + b_ref[...]
    o_ref[...] = jnp.maximum(z, 0.0).astype(o_ref.dtype)


def _head_body(a_ref, w1_ref, b1_ref, s1_ref, t1_ref,
               w2_ref, b2_ref, s2_ref, t2_ref, o_ref):
    f32 = jnp.float32
    z1 = jnp.dot(a_ref[...], w1_ref[...], preferred_element_type=f32) + b1_ref[...]
    y1 = jnp.maximum(z1 * s1_ref[...] + t1_ref[...], 0.0)
    z2 = jnp.dot(y1.astype(jnp.bfloat16), w2_ref[...],
                 preferred_element_type=f32) + b2_ref[...]
    o_ref[...] = jnp.maximum(z2 * s2_ref[...] + t2_ref[...], 0.0)


def kernel(x, stem_w, stem_b, s0b0_w1, s0b0_b1, s0b0_w2, s0b0_b2, s0b0_w3, s0b0_b3, s0b0_wd, s0b0_bd, s0b1_w1, s0b1_b1, s0b1_w2, s0b1_b2, s0b1_w3, s0b1_b3, s0b2_w1, s0b2_b1, s0b2_w2, s0b2_b2, s0b2_w3, s0b2_b3, s1b0_w1, s1b0_b1, s1b0_w2, s1b0_b2, s1b0_w3, s1b0_b3, s1b0_wd, s1b0_bd, s1b1_w1, s1b1_b1, s1b1_w2, s1b1_b2, s1b1_w3, s1b1_b3, s1b2_w1, s1b2_b1, s1b2_w2, s1b2_b2, s1b2_w3, s1b2_b3, s1b3_w1, s1b3_b1, s1b3_w2, s1b3_b2, s1b3_w3, s1b3_b3, s2b0_w1, s2b0_b1, s2b0_w2, s2b0_b2, s2b0_w3, s2b0_b3, s2b0_wd, s2b0_bd, s2b1_w1, s2b1_b1, s2b1_w2, s2b1_b2, s2b1_w3, s2b1_b3, s2b2_w1, s2b2_b1, s2b2_w2, s2b2_b2, s2b2_w3, s2b2_b3, s2b3_w1, s2b3_b1, s2b3_w2, s2b3_b2, s2b3_w3, s2b3_b3, s2b4_w1, s2b4_b1, s2b4_w2, s2b4_b2, s2b4_w3, s2b4_b3, s2b5_w1, s2b5_b1, s2b5_w2, s2b5_b2, s2b5_w3, s2b5_b3, s3b0_w1, s3b0_b1, s3b0_w2, s3b0_b2, s3b0_w3, s3b0_b3, s3b0_wd, s3b0_bd, s3b1_w1, s3b1_b1, s3b1_w2, s3b1_b2, s3b1_w3, s3b1_b3, s3b2_w1, s3b2_b1, s3b2_w2, s3b2_b2, s3b2_w3, s3b2_b3, fc1_w, fc1_b, fc2_w, fc2_b, hbn1_s, hbn1_h, hbn2_s, hbn2_h):
    n = x.shape[0]
    f32, bf16 = jnp.float32, jnp.bfloat16

    xh = jnp.transpose(x, (0, 2, 3, 1)).astype(bf16)
    xp = jnp.pad(xh, ((0, 0), (3, 3), (3, 3), (0, 0)))
    cols = [xp[:, di:di + 64:2, dj:dj + 64:2, :]
            for di in range(7) for dj in range(7)]
    patches = jnp.concatenate(cols, axis=-1).reshape(n * 1024, 147)

    gs = _GS if n % _GS == 0 else 1
    stem = pl.pallas_call(
        _stem_body,
        out_shape=jax.ShapeDtypeStruct((n * 1024, 128), bf16),
        grid=(n // gs,),
        in_specs=[pl.BlockSpec((gs * 1024, 147), lambda i: (i, 0)),
                  pl.BlockSpec(stem_w.shape, lambda i: (0, 0)),
                  pl.BlockSpec(stem_b.shape, lambda i: (0, 0))],
        out_specs=pl.BlockSpec((gs * 1024, 128), lambda i: (i, 0)),
        compiler_params=pltpu.CompilerParams(dimension_semantics=("parallel",)),
    )(patches, stem_w, stem_b)

    sm = stem.reshape(n, 32, 32, 128)
    smp = jnp.pad(sm, ((0, 0), (1, 1), (1, 1), (0, 0)),
                  constant_values=-jnp.inf)
    feat = None
    for di in range(3):
        for dj in range(3):
            s = smp[:, di:di + 32:2, dj:dj + 32:2, :]
            feat = s if feat is None else jnp.maximum(feat, s)
    feat = feat.reshape(n * 256, 128)

    wargs = [s0b0_w1, s0b0_b1, s0b0_w2, s0b0_b2, s0b0_w3, s0b0_b3, s0b0_wd, s0b0_bd,
             s0b1_w1, s0b1_b1, s0b1_w2, s0b1_b2, s0b1_w3, s0b1_b3,
             s0b2_w1, s0b2_b1, s0b2_w2, s0b2_b2, s0b2_w3, s0b2_b3,
             s1b0_w1, s1b0_b1, s1b0_w2, s1b0_b2, s1b0_w3, s1b0_b3, s1b0_wd, s1b0_bd,
             s1b1_w1, s1b1_b1, s1b1_w2, s1b1_b2, s1b1_w3, s1b1_b3,
             s1b2_w1, s1b2_b1, s1b2_w2, s1b2_b2, s1b2_w3, s1b2_b3,
             s1b3_w1, s1b3_b1, s1b3_w2, s1b3_b2, s1b3_w3, s1b3_b3,
             s2b0_w1, s2b0_b1, s2b0_w2, s2b0_b2, s2b0_w3, s2b0_b3, s2b0_wd, s2b0_bd,
             s2b1_w1, s2b1_b1, s2b1_w2, s2b1_b2, s2b1_w3, s2b1_b3,
             s2b2_w1, s2b2_b1, s2b2_w2, s2b2_b2, s2b2_w3, s2b2_b3,
             s2b3_w1, s2b3_b1, s2b3_w2, s2b3_b2, s2b3_w3, s2b3_b3,
             s2b4_w1, s2b4_b1, s2b4_w2, s2b4_b2, s2b4_w3, s2b4_b3,
             s2b5_w1, s2b5_b1, s2b5_w2, s2b5_b2, s2b5_w3, s2b5_b3,
             s3b0_w1, s3b0_b1, s3b0_w2, s3b0_b2, s3b0_w3, s3b0_b3, s3b0_wd, s3b0_bd,
             s3b1_w1, s3b1_b1, s3b1_w2, s3b1_b2, s3b1_w3, s3b1_b3,
             s3b2_w1, s3b2_b1, s3b2_w2, s3b2_b2, s3b2_w3, s3b2_b3]

    g = _G if n % _G == 0 else 1
    scratch = [pltpu.VMEM((g * 256 + 2 * _halo(16), 8), f32),
               pltpu.VMEM((g * 256 + 2 * _halo(16), 16), f32),
               pltpu.VMEM((g * 64 + 2 * _halo(8), 32), f32),
               pltpu.VMEM((g * 16 + 2 * _halo(4), 64), f32)]
    res = pl.pallas_call(
        _stages_body,
        out_shape=jax.ShapeDtypeStruct((n * 4, 256), bf16),
        grid=(n // g,),
        in_specs=[pl.BlockSpec((g * 256, 128), lambda i: (i, 0))] +
                 [pl.BlockSpec(a.shape, lambda i: (0, 0)) for a in wargs],
        out_specs=pl.BlockSpec((g * 4, 256), lambda i: (i, 0)),
        scratch_shapes=scratch,
        compiler_params=pltpu.CompilerParams(dimension_semantics=("parallel",)),
    )(feat, *wargs)

    a = res.reshape(n, 2, 2, 256).transpose(0, 3, 1, 2).reshape(n * 4, 256)
    s1 = jnp.tile(hbn1_s, n).reshape(n * 4, 1).astype(f32)
    t1 = jnp.tile(hbn1_h, n).reshape(n * 4, 1).astype(f32)
    s2 = jnp.tile(hbn2_s, n).reshape(n * 4, 1).astype(f32)
    t2 = jnp.tile(hbn2_h, n).reshape(n * 4, 1).astype(f32)

    rows = n * 4
    gh = 2 if rows % 16 == 0 else 1
    rb = rows // gh
    out = pl.pallas_call(
        _head_body,
        out_shape=jax.ShapeDtypeStruct((rows, 128), f32),
        grid=(gh,),
        in_specs=[pl.BlockSpec((rb, 256), lambda i: (i, 0)),
                  pl.BlockSpec(fc1_w.shape, lambda i: (0, 0)),
                  pl.BlockSpec(fc1_b.shape, lambda i: (0, 0)),
                  pl.BlockSpec((rb, 1), lambda i: (i, 0)),
                  pl.BlockSpec((rb, 1), lambda i: (i, 0)),
                  pl.BlockSpec(fc2_w.shape, lambda i: (0, 0)),
                  pl.BlockSpec(fc2_b.shape, lambda i: (0, 0)),
                  pl.BlockSpec((rb, 1), lambda i: (i, 0)),
                  pl.BlockSpec((rb, 1), lambda i: (i, 0))],
        out_specs=pl.BlockSpec((rb, 128), lambda i: (i, 0)),
        compiler_params=pltpu.CompilerParams(dimension_semantics=("parallel",)),
    )(a, fc1_w, fc1_b, s1, t1, fc2_w, fc2_b, s2, t2)

    return out[:, :32].reshape(n, 4, 32)
